# baseline (speedup 1.0000x reference)
import functools

import jax
import jax.numpy as jnp
from jax import lax
from jax.experimental import pallas as pl
from jax.experimental.pallas import tpu as pltpu
from jax.experimental.pallas import tpu_sc as plsc

B = 4096
ED = 32
NC = 2
NS = 16
NW = NC * NS
BPW = B // NW
_C = 32

_R = 256
_NBLK = B // _R


def _sc_gather_body(t0, t1, t2, t3, t4, t5, i0, i1, i2, i3, i4, i5,
                    o0, o1, o2, o3, o4, o5,
                    idx_v, idx4_v, buf_v, rows_v, sem):
    wid = lax.axis_index("s") * NC + lax.axis_index("c")
    base = wid * BPW
    for tab, idx, out in ((t0, i0, o0), (t1, i1, o1), (t2, i2, o2),
                          (t3, i3, o3), (t4, i4, o4), (t5, i5, o5)):
        pltpu.sync_copy(idx.at[pl.ds(base, BPW)], idx_v)
        for j in range(BPW // 16):
            idx4_v[pl.ds(j * 16, 16)] = lax.shift_right_logical(
                idx_v[pl.ds(j * 16, 16)], 2)
        pltpu.async_copy(tab.at[idx4_v], buf_v, sem).wait()

        def row(k, _):
            v = idx_v[pl.ds(k * 16, 16)]
            for j in range(16):
                s = (v[j] & 3) * ED
                r = k * 16 + j
                rows_v[r, pl.ds(0, 16)] = buf_v[r, pl.ds(s, 16)]
                rows_v[r, pl.ds(16, 16)] = buf_v[r, pl.ds(s + 16, 16)]
            return 0

        lax.fori_loop(0, BPW // 16, row, 0)
        pltpu.sync_copy(rows_v, out.at[pl.ds(base, BPW)])


def _sc_gather(tables, idxs):
    mesh = plsc.VectorSubcoreMesh(core_axis_name="c", subcore_axis_name="s")
    fn = functools.partial(
        pl.kernel,
        mesh=mesh,
        out_type=[jax.ShapeDtypeStruct((B, ED), jnp.float32)] * 6,
        scratch_types=[
            pltpu.VMEM((BPW,), jnp.int32),
            pltpu.VMEM((BPW,), jnp.int32),
            pltpu.VMEM((BPW, 4 * ED), jnp.float32),
            pltpu.VMEM((BPW, ED), jnp.float32),
            pltpu.SemaphoreType.DMA,
        ],
    )(_sc_gather_body)
    return fn(*tables, *idxs)


def _tc_body(g0, g1, g2, g3, g4f, g5f, lab,
             Wu1, bu1, Wu2, bu2, Wi1, bi1, Wi2, bi2,
             out, item_scr):
    @pl.when(pl.program_id(0) == 0)
    def _():
        wi1 = Wi1[...]
        ih = jnp.maximum(
            g4f[...] @ wi1[:ED, :] + g5f[...] @ wi1[ED:, :] + bi1[...], 0.0)
        item_scr[...] = jnp.maximum(ih @ Wi2[...] + bi2[...], 0.0)

    wu1 = Wu1[...]
    uh = jnp.maximum(
        g0[...] @ wu1[0 * ED:1 * ED, :] + g1[...] @ wu1[1 * ED:2 * ED, :]
        + g2[...] @ wu1[2 * ED:3 * ED, :] + g3[...] @ wu1[3 * ED:4 * ED, :]
        + bu1[...], 0.0)
    uo = jnp.maximum(uh @ Wu2[...] + bu2[...], 0.0)
    logits = lax.dot_general(uo, item_scr[...],
                             (((1,), (1,)), ((), ())))
    m = jnp.max(logits, axis=1, keepdims=True)
    s = jnp.sum(jnp.exp(logits - m), axis=1, keepdims=True)
    lse = m + jnp.log(s)
    cols = lax.broadcasted_iota(jnp.int32, (_R, B), 1)
    tgt = jnp.sum(jnp.where(cols == lab[...], logits, 0.0), axis=1,
                  keepdims=True)
    out[...] = lse - tgt


def _tc_dense(g, labels, Wu1, bu1, Wu2, bu2, Wi1, bi1, Wi2, bi2):
    blk = lambda i: (i, 0)
    full = lambda i: (0, 0)
    return pl.pallas_call(
        _tc_body,
        grid=(_NBLK,),
        in_specs=[
            pl.BlockSpec((_R, ED), blk), pl.BlockSpec((_R, ED), blk),
            pl.BlockSpec((_R, ED), blk), pl.BlockSpec((_R, ED), blk),
            pl.BlockSpec((B, ED), full), pl.BlockSpec((B, ED), full),
            pl.BlockSpec((_R, 1), blk),
            pl.BlockSpec((128, 64), full), pl.BlockSpec((1, 64), full),
            pl.BlockSpec((64, 32), full), pl.BlockSpec((1, 32), full),
            pl.BlockSpec((64, 64), full), pl.BlockSpec((1, 64), full),
            pl.BlockSpec((64, 32), full), pl.BlockSpec((1, 32), full),
        ],
        out_specs=pl.BlockSpec((_R, 1), blk),
        out_shape=jax.ShapeDtypeStruct((B, 1), jnp.float32),
        scratch_shapes=[pltpu.VMEM((B, ED), jnp.float32)],
    )(g[0], g[1], g[2], g[3], g[4], g[5], labels,
      Wu1, bu1, Wu2, bu2, Wi1, bi1, Wi2, bi2)


def kernel(user_id, user_city, user_device, user_age, item_id, item_cate,
           labels, E_user_id, E_user_city, E_user_device, E_user_age,
           E_item_id, E_item_cate, Wu1, bu1, Wu2, bu2, Wi1, bi1, Wi2, bi2):
    idxs = [x[:, 0].astype(jnp.int32) for x in
            (user_id, user_city, user_device, user_age, item_id, item_cate)]
    tables = tuple(t.reshape(t.shape[0] // 4, 4 * ED) for t in
                   (E_user_id, E_user_city, E_user_device, E_user_age,
                    E_item_id, E_item_cate))
    g = _sc_gather(tables, idxs)
    loss = _tc_dense(g, labels.astype(jnp.int32),
                     Wu1, bu1.reshape(1, -1), Wu2, bu2.reshape(1, -1),
                     Wi1, bi1.reshape(1, -1), Wi2, bi2.reshape(1, -1))
    return loss[:, 0]


# speedup vs baseline: 1.0174x; 1.0174x over previous
import functools

import jax
import jax.numpy as jnp
from jax import lax
from jax.experimental import pallas as pl
from jax.experimental.pallas import tpu as pltpu
from jax.experimental.pallas import tpu_sc as plsc

B = 4096
ED = 32
NC = 2
NS = 16
NW = NC * NS
BPW = B // NW
_C = 32

_R = 256
_NBLK = B // _R


def _sc_gather_body(t0, t1, t2, t3, t4, t5, i0, i1, i2, i3, i4, i5,
                    o0, o1, o2, o3, o4, o5,
                    idx_v, rows_v, sem):
    wid = lax.axis_index("s") * NC + lax.axis_index("c")
    base = wid * BPW
    for tab, idx, out in ((t0, i0, o0), (t1, i1, o1), (t2, i2, o2),
                          (t3, i3, o3), (t4, i4, o4), (t5, i5, o5)):
        pltpu.sync_copy(idx.at[pl.ds(base, BPW)], idx_v)

        def fire(k, _):
            v = idx_v[pl.ds(k * 16, 16)] * ED
            for j in range(16):
                off = pl.multiple_of(v[j], ED)
                pltpu.async_copy(tab.at[pl.ds(off, ED)],
                                 rows_v.at[k * 16 + j], sem)
            return 0

        lax.fori_loop(0, BPW // 16, fire, 0)
        def drain(k, _):
            pltpu.make_async_copy(tab.at[pl.ds(0, ED)],
                                  rows_v.at[0], sem).wait()
            return 0

        lax.fori_loop(0, BPW, drain, 0)
        pltpu.sync_copy(rows_v, out.at[pl.ds(base, BPW)])


def _sc_gather(tables, idxs):
    mesh = plsc.VectorSubcoreMesh(core_axis_name="c", subcore_axis_name="s")
    fn = functools.partial(
        pl.kernel,
        mesh=mesh,
        out_type=[jax.ShapeDtypeStruct((B, ED), jnp.float32)] * 6,
        scratch_types=[
            pltpu.VMEM((BPW,), jnp.int32),
            pltpu.VMEM((BPW, ED), jnp.float32),
            pltpu.SemaphoreType.DMA,
        ],
    )(_sc_gather_body)
    return fn(*tables, *idxs)


def _tc_body(g0, g1, g2, g3, g4f, g5f, lab,
             Wu1, bu1, Wu2, bu2, Wi1, bi1, Wi2, bi2,
             out, item_scr):
    @pl.when(pl.program_id(0) == 0)
    def _():
        wi1 = Wi1[...]
        ih = jnp.maximum(
            g4f[...] @ wi1[:ED, :] + g5f[...] @ wi1[ED:, :] + bi1[...], 0.0)
        item_scr[...] = jnp.maximum(ih @ Wi2[...] + bi2[...], 0.0)

    wu1 = Wu1[...]
    uh = jnp.maximum(
        g0[...] @ wu1[0 * ED:1 * ED, :] + g1[...] @ wu1[1 * ED:2 * ED, :]
        + g2[...] @ wu1[2 * ED:3 * ED, :] + g3[...] @ wu1[3 * ED:4 * ED, :]
        + bu1[...], 0.0)
    uo = jnp.maximum(uh @ Wu2[...] + bu2[...], 0.0)
    logits = lax.dot_general(uo, item_scr[...],
                             (((1,), (1,)), ((), ())))
    m = jnp.max(logits, axis=1, keepdims=True)
    s = jnp.sum(jnp.exp(logits - m), axis=1, keepdims=True)
    lse = m + jnp.log(s)
    cols = lax.broadcasted_iota(jnp.int32, (_R, B), 1)
    tgt = jnp.sum(jnp.where(cols == lab[...], logits, 0.0), axis=1,
                  keepdims=True)
    out[...] = lse - tgt


def _tc_dense(g, labels, Wu1, bu1, Wu2, bu2, Wi1, bi1, Wi2, bi2):
    blk = lambda i: (i, 0)
    full = lambda i: (0, 0)
    return pl.pallas_call(
        _tc_body,
        grid=(_NBLK,),
        in_specs=[
            pl.BlockSpec((_R, ED), blk), pl.BlockSpec((_R, ED), blk),
            pl.BlockSpec((_R, ED), blk), pl.BlockSpec((_R, ED), blk),
            pl.BlockSpec((B, ED), full), pl.BlockSpec((B, ED), full),
            pl.BlockSpec((_R, 1), blk),
            pl.BlockSpec((128, 64), full), pl.BlockSpec((1, 64), full),
            pl.BlockSpec((64, 32), full), pl.BlockSpec((1, 32), full),
            pl.BlockSpec((64, 64), full), pl.BlockSpec((1, 64), full),
            pl.BlockSpec((64, 32), full), pl.BlockSpec((1, 32), full),
        ],
        out_specs=pl.BlockSpec((_R, 1), blk),
        out_shape=jax.ShapeDtypeStruct((B, 1), jnp.float32),
        scratch_shapes=[pltpu.VMEM((B, ED), jnp.float32)],
    )(g[0], g[1], g[2], g[3], g[4], g[5], labels,
      Wu1, bu1, Wu2, bu2, Wi1, bi1, Wi2, bi2)


def kernel(user_id, user_city, user_device, user_age, item_id, item_cate,
           labels, E_user_id, E_user_city, E_user_device, E_user_age,
           E_item_id, E_item_cate, Wu1, bu1, Wu2, bu2, Wi1, bi1, Wi2, bi2):
    idxs = [x[:, 0].astype(jnp.int32) for x in
            (user_id, user_city, user_device, user_age, item_id, item_cate)]
    tables = tuple(t.reshape(-1) for t in
                   (E_user_id, E_user_city, E_user_device, E_user_age,
                    E_item_id, E_item_cate))
    g = _sc_gather(tables, idxs)
    loss = _tc_dense(g, labels.astype(jnp.int32),
                     Wu1, bu1.reshape(1, -1), Wu2, bu2.reshape(1, -1),
                     Wi1, bi1.reshape(1, -1), Wi2, bi2.reshape(1, -1))
    return loss[:, 0]


# speedup vs baseline: 4.9513x; 4.8667x over previous
import functools

import jax
import jax.numpy as jnp
from jax import lax
from jax.experimental import pallas as pl
from jax.experimental.pallas import tpu as pltpu
from jax.experimental.pallas import tpu_sc as plsc

B = 4096
ED = 32
NC = 2
NS = 16
NW = NC * NS
BPW = B // NW

_R = 256
_NBLK = B // _R


_CH = 2048


def _splat(c):
    return jnp.full((16,), c, jnp.int32)


def _gather_block_table(tab, idx_v, rowsT_v, buf_v, sem, V):
    def batch(k, _):
        v = idx_v[pl.ds(k * 16, 16)]
        for j in range(16):
            off = pl.multiple_of((v[j] >> 7) << 7, 128)
            pltpu.async_copy(tab.at[:, pl.ds(off, 128)],
                             buf_v.at[:, pl.ds(j * 128, 128)], sem)
        pltpu.make_async_copy(tab.at[:, pl.ds(0, 16 * 128)],
                              buf_v.at[:, pl.ds(0, 16 * 128)], sem).wait()
        lane_v = (v & 127) + lax.iota(jnp.int32, 16) * 128
        for c in range(ED):
            g = plsc.load_gather(buf_v, [_splat(c), lane_v])
            rowsT_v[c, pl.ds(k * 16, 16)] = g
        return 0

    lax.fori_loop(0, BPW // 16, batch, 0)


def _gather_chunked_table(tab, idx_v, rowsT_v, buf_v, sem, V):
    nch = (V + _CH - 1) // _CH
    for ci in range(nch):
        lo = ci * _CH
        w = min(_CH, V - lo)
        pltpu.async_copy(tab.at[:, pl.ds(lo, w)],
                         buf_v.at[:, pl.ds(0, w)], sem).wait()

        def batch(k, _):
            v = idx_v[pl.ds(k * 16, 16)]
            m = (v >= lo) & (v < lo + w)
            l_v = jnp.where(m, v - lo, 0)
            for c in range(ED):
                g = plsc.load_gather(buf_v, [_splat(c), l_v])
                cur = rowsT_v[c, pl.ds(k * 16, 16)]
                rowsT_v[c, pl.ds(k * 16, 16)] = jnp.where(m, g, cur)
            return 0

        lax.fori_loop(0, BPW // 16, batch, 0)


def _gather_staged_table(tab, idx_v, rowsT_v, buf_v, sem, V):
    pltpu.async_copy(tab.at[:, pl.ds(0, V)],
                     buf_v.at[:, pl.ds(0, V)], sem).wait()

    def batch(k, _):
        v = idx_v[pl.ds(k * 16, 16)]
        for c in range(ED):
            g = plsc.load_gather(buf_v, [_splat(c), v])
            rowsT_v[c, pl.ds(k * 16, 16)] = g
        return 0

    lax.fori_loop(0, BPW // 16, batch, 0)


def _sc_gather_body(t0, t1, t2, t3, t4, t5, i0, i1, i2, i3, i4, i5,
                    o0, o1, o2, o3, o4, o5,
                    idx_v, rowsT_v, buf_v, sem):
    wid = lax.axis_index("s") * NC + lax.axis_index("c")
    base = wid * BPW
    plans = ((t0, i0, o0, _gather_block_table, 1000000),
             (t1, i1, o1, _gather_chunked_table, 10112),
             (t2, i2, o2, _gather_staged_table, 1024),
             (t3, i3, o3, _gather_staged_table, 128),
             (t4, i4, o4, _gather_block_table, 1000000),
             (t5, i5, o5, _gather_chunked_table, 10112))
    for tab, idx, out, fn, V in plans:
        pltpu.sync_copy(idx.at[pl.ds(base, BPW)], idx_v)
        fn(tab, idx_v, rowsT_v, buf_v, sem, V)
        pltpu.sync_copy(rowsT_v, out.at[:, pl.ds(base, BPW)])


def _sc_gather(tables, idxs):
    mesh = plsc.VectorSubcoreMesh(core_axis_name="c", subcore_axis_name="s")
    fn = functools.partial(
        pl.kernel,
        mesh=mesh,
        compiler_params=pltpu.CompilerParams(disable_bounds_checks=True, needs_layout_passes=False),
        out_type=[jax.ShapeDtypeStruct((ED, B), jnp.float32)] * 6,
        scratch_types=[
            pltpu.VMEM((BPW,), jnp.int32),
            pltpu.VMEM((ED, BPW), jnp.float32),
            pltpu.VMEM((ED, _CH), jnp.float32),
            pltpu.SemaphoreType.DMA,
        ],
    )(_sc_gather_body)
    return fn(*tables, *idxs)


def _tc_body(g0, g1, g2, g3, g4f, g5f, lab,
             Wu1, bu1, Wu2, bu2, Wi1, bi1, Wi2, bi2,
             out, item_scr):
    dnT = (((0,), (0,)), ((), ()))

    @pl.when(pl.program_id(0) == 0)
    def _():
        wi1 = Wi1[...]
        ih = jnp.maximum(
            lax.dot_general(g4f[...], wi1[:ED, :], dnT)
            + lax.dot_general(g5f[...], wi1[ED:, :], dnT) + bi1[...], 0.0)
        item_scr[...] = jnp.maximum(ih @ Wi2[...] + bi2[...], 0.0)

    wu1 = Wu1[...]
    uh = jnp.maximum(
        lax.dot_general(g0[...], wu1[0 * ED:1 * ED, :], dnT)
        + lax.dot_general(g1[...], wu1[1 * ED:2 * ED, :], dnT)
        + lax.dot_general(g2[...], wu1[2 * ED:3 * ED, :], dnT)
        + lax.dot_general(g3[...], wu1[3 * ED:4 * ED, :], dnT)
        + bu1[...], 0.0)
    uo = jnp.maximum(uh @ Wu2[...] + bu2[...], 0.0)
    logits = lax.dot_general(uo, item_scr[...],
                             (((1,), (1,)), ((), ())))
    m = jnp.max(logits, axis=1, keepdims=True)
    s = jnp.sum(jnp.exp(logits - m), axis=1, keepdims=True)
    lse = m + jnp.log(s)
    cols = lax.broadcasted_iota(jnp.int32, (_R, B), 1)
    tgt = jnp.sum(jnp.where(cols == lab[...], logits, 0.0), axis=1,
                  keepdims=True)
    out[...] = lse - tgt


def _tc_dense(g, labels, Wu1, bu1, Wu2, bu2, Wi1, bi1, Wi2, bi2):
    blk = lambda i: (i, 0)
    col = lambda i: (0, i)
    full = lambda i: (0, 0)
    return pl.pallas_call(
        _tc_body,
        grid=(_NBLK,),
        in_specs=[
            pl.BlockSpec((ED, _R), col), pl.BlockSpec((ED, _R), col),
            pl.BlockSpec((ED, _R), col), pl.BlockSpec((ED, _R), col),
            pl.BlockSpec((ED, B), full), pl.BlockSpec((ED, B), full),
            pl.BlockSpec((_R, 1), blk),
            pl.BlockSpec((128, 64), full), pl.BlockSpec((1, 64), full),
            pl.BlockSpec((64, 32), full), pl.BlockSpec((1, 32), full),
            pl.BlockSpec((64, 64), full), pl.BlockSpec((1, 64), full),
            pl.BlockSpec((64, 32), full), pl.BlockSpec((1, 32), full),
        ],
        out_specs=pl.BlockSpec((_R, 1), blk),
        out_shape=jax.ShapeDtypeStruct((B, 1), jnp.float32),
        scratch_shapes=[pltpu.VMEM((B, ED), jnp.float32)],
    )(g[0], g[1], g[2], g[3], g[4], g[5], labels,
      Wu1, bu1, Wu2, bu2, Wi1, bi1, Wi2, bi2)


def kernel(user_id, user_city, user_device, user_age, item_id, item_cate,
           labels, E_user_id, E_user_city, E_user_device, E_user_age,
           E_item_id, E_item_cate, Wu1, bu1, Wu2, bu2, Wi1, bi1, Wi2, bi2):
    idxs = [x[:, 0].astype(jnp.int32) for x in
            (user_id, user_city, user_device, user_age, item_id, item_cate)]
    def padT(t):
        v = t.shape[0]
        vp = -(-v // 128) * 128
        return jnp.pad(t, ((0, vp - v), (0, 0))).T

    tables = (E_user_id.T, padT(E_user_city), padT(E_user_device),
              padT(E_user_age), E_item_id.T, padT(E_item_cate))
    g = _sc_gather(tables, idxs)
    loss = _tc_dense(g, labels.astype(jnp.int32),
                     Wu1, bu1.reshape(1, -1), Wu2, bu2.reshape(1, -1),
                     Wi1, bi1.reshape(1, -1), Wi2, bi2.reshape(1, -1))
    return loss[:, 0]


# speedup vs baseline: 4.9516x; 1.0001x over previous
import functools

import jax
import jax.numpy as jnp
from jax import lax
from jax.experimental import pallas as pl
from jax.experimental.pallas import tpu as pltpu
from jax.experimental.pallas import tpu_sc as plsc

B = 4096
ED = 32
NC = 2
NS = 16
NW = NC * NS
BPW = B // NW

_R = 256
_NBLK = B // _R


_CH = 896


def _splat(c):
    return jnp.full((16,), c, jnp.int32)


def _gather_block_table(tab, idx_v, rowsT_v, buf_v, sem, V):
    def batch(k, _):
        v = idx_v[pl.ds(k * 16, 16)]
        for j in range(16):
            off = pl.multiple_of((v[j] >> 7) << 7, 128)
            pltpu.async_copy(tab.at[:, pl.ds(off, 128)],
                             buf_v.at[:, pl.ds(j * 128, 128)], sem)
        pltpu.make_async_copy(tab.at[:, pl.ds(0, 16 * 128)],
                              buf_v.at[:, pl.ds(0, 16 * 128)], sem).wait()
        lane_v = (v & 127) + lax.iota(jnp.int32, 16) * 128
        for c in range(ED):
            g = plsc.load_gather(buf_v, [_splat(c), lane_v])
            rowsT_v[c, pl.ds(k * 16, 16)] = g
        return 0

    lax.fori_loop(0, BPW // 16, batch, 0)


def _gather_chunked_table(tab, idx_v, rowsT_v, cbuf, sem0, sem1, V):
    nch = V // _CH
    pltpu.async_copy(tab.at[:, pl.ds(0, _CH)],
                     cbuf.at[:, pl.ds(0, _CH)], sem0)

    def chunk(ci, _):
        lo = ci * _CH
        even = (ci & 1) == 0

        @pl.when(even)
        def _():
            pltpu.make_async_copy(tab.at[:, pl.ds(0, _CH)],
                                  cbuf.at[:, pl.ds(0, _CH)], sem0).wait()

        @pl.when(jnp.logical_not(even))
        def _():
            pltpu.make_async_copy(tab.at[:, pl.ds(0, _CH)],
                                  cbuf.at[:, pl.ds(_CH, _CH)], sem1).wait()

        nxt = pl.multiple_of(lo + _CH, 128)

        @pl.when(even & (ci + 1 < nch))
        def _():
            pltpu.async_copy(tab.at[:, pl.ds(nxt, _CH)],
                             cbuf.at[:, pl.ds(_CH, _CH)], sem1)

        @pl.when(jnp.logical_not(even) & (ci + 1 < nch))
        def _():
            pltpu.async_copy(tab.at[:, pl.ds(nxt, _CH)],
                             cbuf.at[:, pl.ds(0, _CH)], sem0)

        o_v = jnp.where(jnp.full((16,), even, jnp.bool_),
                        jnp.full((16,), 0, jnp.int32),
                        jnp.full((16,), _CH, jnp.int32))
        later = lax.broadcast(ci > 0, (16,))

        def batch(k, _):
            v = idx_v[pl.ds(k * 16, 16)]
            m = (v >= lo) & (v < lo + _CH)
            l_v = jnp.where(m, v - lo, 0) + o_v
            for c in range(ED):
                g = plsc.load_gather(cbuf, [_splat(c), l_v])
                cur = jnp.where(later,
                                rowsT_v[c, pl.ds(k * 16, 16)], 0.0)
                rowsT_v[c, pl.ds(k * 16, 16)] = jnp.where(m, g, cur)
            return 0

        lax.fori_loop(0, BPW // 16, batch, 0)
        return 0

    lax.fori_loop(0, nch, chunk, 0)


def _sc_gather_body(t0, t1, t2, t3, t4, t5, i0, i1, i2, i3, i4, i5,
                    o0, o1, o2, o3, o4, o5,
                    idx_v, rowsT_v, buf_v, cbuf, sem, sem0, sem1):
    wid = lax.axis_index("s") * NC + lax.axis_index("c")
    base = wid * BPW
    plans = ((t0, i0, o0, True, 1000000),
             (t4, i4, o4, True, 1000000),
             (t1, i1, o1, False, 10752),
             (t2, i2, o2, False, 1792),
             (t3, i3, o3, False, 896),
             (t5, i5, o5, False, 10752))
    for tab, idx, out, big, V in plans:
        pltpu.sync_copy(idx.at[pl.ds(base, BPW)], idx_v)
        if big:
            _gather_block_table(tab, idx_v, rowsT_v, buf_v, sem, V)
        else:
            _gather_chunked_table(tab, idx_v, rowsT_v, cbuf,
                                  sem0, sem1, V)
        pltpu.sync_copy(rowsT_v, out.at[:, pl.ds(base, BPW)])


def _sc_gather(tables, idxs):
    mesh = plsc.VectorSubcoreMesh(core_axis_name="c", subcore_axis_name="s")
    fn = functools.partial(
        pl.kernel,
        mesh=mesh,
        compiler_params=pltpu.CompilerParams(disable_bounds_checks=True, needs_layout_passes=False),
        out_type=[jax.ShapeDtypeStruct((ED, B), jnp.float32)] * 6,
        scratch_types=[
            pltpu.VMEM((BPW,), jnp.int32),
            pltpu.VMEM((ED, BPW), jnp.float32),
            pltpu.VMEM((ED, 16 * 128), jnp.float32),
            pltpu.VMEM((ED, 2 * _CH), jnp.float32),
            pltpu.SemaphoreType.DMA,
            pltpu.SemaphoreType.DMA,
            pltpu.SemaphoreType.DMA,
        ],
    )(_sc_gather_body)
    return fn(*tables, *idxs)


def _tc_body(g0, g1, g2, g3, g4f, g5f, lab,
             Wu1, bu1, Wu2, bu2, Wi1, bi1, Wi2, bi2,
             out, item_scr):
    dnT = (((0,), (0,)), ((), ()))

    @pl.when(pl.program_id(0) == 0)
    def _():
        wi1 = Wi1[...]
        ih = jnp.maximum(
            lax.dot_general(g4f[...], wi1[:ED, :], dnT)
            + lax.dot_general(g5f[...], wi1[ED:, :], dnT) + bi1[...], 0.0)
        item_scr[...] = jnp.maximum(ih @ Wi2[...] + bi2[...], 0.0)

    wu1 = Wu1[...]
    uh = jnp.maximum(
        lax.dot_general(g0[...], wu1[0 * ED:1 * ED, :], dnT)
        + lax.dot_general(g1[...], wu1[1 * ED:2 * ED, :], dnT)
        + lax.dot_general(g2[...], wu1[2 * ED:3 * ED, :], dnT)
        + lax.dot_general(g3[...], wu1[3 * ED:4 * ED, :], dnT)
        + bu1[...], 0.0)
    uo = jnp.maximum(uh @ Wu2[...] + bu2[...], 0.0)
    logits = lax.dot_general(uo, item_scr[...],
                             (((1,), (1,)), ((), ())))
    m = jnp.max(logits, axis=1, keepdims=True)
    s = jnp.sum(jnp.exp(logits - m), axis=1, keepdims=True)
    lse = m + jnp.log(s)
    cols = lax.broadcasted_iota(jnp.int32, (_R, B), 1)
    tgt = jnp.sum(jnp.where(cols == lab[...], logits, 0.0), axis=1,
                  keepdims=True)
    out[...] = lse - tgt


def _tc_dense(g, labels, Wu1, bu1, Wu2, bu2, Wi1, bi1, Wi2, bi2):
    blk = lambda i: (i, 0)
    col = lambda i: (0, i)
    full = lambda i: (0, 0)
    return pl.pallas_call(
        _tc_body,
        grid=(_NBLK,),
        in_specs=[
            pl.BlockSpec((ED, _R), col), pl.BlockSpec((ED, _R), col),
            pl.BlockSpec((ED, _R), col), pl.BlockSpec((ED, _R), col),
            pl.BlockSpec((ED, B), full), pl.BlockSpec((ED, B), full),
            pl.BlockSpec((_R, 1), blk),
            pl.BlockSpec((128, 64), full), pl.BlockSpec((1, 64), full),
            pl.BlockSpec((64, 32), full), pl.BlockSpec((1, 32), full),
            pl.BlockSpec((64, 64), full), pl.BlockSpec((1, 64), full),
            pl.BlockSpec((64, 32), full), pl.BlockSpec((1, 32), full),
        ],
        out_specs=pl.BlockSpec((_R, 1), blk),
        out_shape=jax.ShapeDtypeStruct((B, 1), jnp.float32),
        scratch_shapes=[pltpu.VMEM((B, ED), jnp.float32)],
    )(g[0], g[1], g[2], g[3], g[4], g[5], labels,
      Wu1, bu1, Wu2, bu2, Wi1, bi1, Wi2, bi2)


def kernel(user_id, user_city, user_device, user_age, item_id, item_cate,
           labels, E_user_id, E_user_city, E_user_device, E_user_age,
           E_item_id, E_item_cate, Wu1, bu1, Wu2, bu2, Wi1, bi1, Wi2, bi2):
    idxs = [x[:, 0].astype(jnp.int32) for x in
            (user_id, user_city, user_device, user_age, item_id, item_cate)]
    def padT(t):
        v = t.shape[0]
        vp = -(-v // _CH) * _CH
        return jnp.pad(t, ((0, vp - v), (0, 0))).T

    tables = (E_user_id.T, padT(E_user_city), padT(E_user_device),
              padT(E_user_age), E_item_id.T, padT(E_item_cate))
    g = _sc_gather(tables, idxs)
    loss = _tc_dense(g, labels.astype(jnp.int32),
                     Wu1, bu1.reshape(1, -1), Wu2, bu2.reshape(1, -1),
                     Wi1, bi1.reshape(1, -1), Wi2, bi2.reshape(1, -1))
    return loss[:, 0]
